# baseline (speedup 1.0000x reference)
import jax
import jax.numpy as jnp
from jax import lax
from jax.experimental import pallas as pl
from jax.experimental.pallas import tpu as pltpu
from jax.experimental.pallas import tpu_sc as plsc

N = 16_000_000
P = 32_000
L = 191
D1 = 25
D2 = 166
UL = P * L
RATIO = 0.5
NC, NS = 2, 16
NW = NC * NS
RANGE = N // NW
CHUNK = 10_000
NCH = RANGE // CHUNK
NCHT = NW * NCH
GB = 128
SB = 2048
FCAP = P + GB


def _sload(ref, i):
    return ref[pl.ds(i, 16)][0]


def _body(x_hbm, r_hbm, sk_hbm, vu_hbm, bounds_hbm, out_hbm,
          r_v, fill_v, idx_v, buf_v, sk_v, vu_v, bounds_v, sem):
    wid = lax.axis_index("s") * NC + lax.axis_index("c")
    base = wid * RANGE
    is_last = wid == NW - 1

    pltpu.sync_copy(r_hbm, r_v.at[pl.ds(0, P)])
    pltpu.sync_copy(bounds_hbm, bounds_v.at[pl.ds(0, NCHT + 1)])

    def first_gt(val):
        def step(_, lh):
            lo, hi = lh
            mid = jnp.minimum((lo + hi) // 2, P - 1)
            go = lo < hi
            pred = _sload(r_v, mid) > val
            lo2 = jnp.where(go & ~pred, mid + 1, lo)
            hi2 = jnp.where(go & pred, mid, hi)
            return lo2, hi2
        lo, _ = lax.fori_loop(0, 15, step, (jnp.int32(0), jnp.int32(P)))
        return lo

    p_lo = jnp.where(is_last, 0, first_gt(base - D2))
    p_hi = jnp.where(is_last, P, first_gt(base + RANGE + D1 - 1))
    wbase = (p_lo // GB) * GB

    nbatch = (p_hi - wbase + GB - 1) // GB

    def gbatch(b, _):
        off = wbase + b * GB
        for g in range(GB // 16):
            rg = r_v[pl.ds(off + g * 16, 16)]
            fi = rg - 26 + jnp.where(rg < 26, N, 0)
            fi = jnp.where(rg == 25, 0, fi)
            idx_v[pl.ds(g * 16, 16)] = jnp.clip(fi, 0, N - 1)
        pltpu.async_copy(x_hbm.at[idx_v], fill_v.at[pl.ds(b * GB, GB)], sem).wait()
        return 0

    lax.fori_loop(0, nbatch, gbatch, 0)

    lane0 = lax.iota(jnp.int32, 16)

    def chunk_body(c, _):
        cb = base + c * CHUNK
        ce = cb + CHUNK
        pltpu.sync_copy(x_hbm.at[pl.ds(cb, CHUNK)], buf_v)

        gcid = wid * NCH + c
        us = _sload(bounds_v, gcid)
        ue = _sload(bounds_v, gcid + 1)
        us0 = us - us % 8
        nsb = (ue - us0 + SB - 1) // SB

        def sbatch(b, _):
            s_off = jnp.minimum(us0 + b * SB, UL - SB)
            s_off = pl.multiple_of(s_off, 8)
            cp1 = pltpu.async_copy(sk_hbm.at[pl.ds(s_off, SB)], sk_v, sem)
            cp2 = pltpu.async_copy(vu_hbm.at[pl.ds(s_off, SB)], vu_v, sem)
            cp1.wait()
            cp2.wait()

            def apply16(t, _):
                k16 = sk_v[pl.ds(t * 16, 16)]
                p16 = vu_v[pl.ds(t * 16, 16)]
                m = (p16 >= 0) & (k16 >= cb) & (k16 < ce)
                fi = jnp.clip(p16 - wbase, 0, FCAP - 1)
                vals = plsc.load_gather(fill_v, [fi])
                plsc.store_scatter(buf_v, [jnp.where(m, k16 - cb, lane0)], vals, mask=m)
                return 0

            lax.fori_loop(0, SB // 16, apply16, 0)
            return 0

        lax.fori_loop(0, nsb, sbatch, 0)

        pltpu.sync_copy(buf_v, out_hbm.at[pl.ds(cb, CHUNK)])
        return 0

    lax.fori_loop(0, NCH, chunk_body, 0)


@jax.jit
def _qtmask_sc(x1d, r32, sk, vu, bounds):
    mesh = plsc.VectorSubcoreMesh(core_axis_name="c", subcore_axis_name="s",
                                  num_cores=NC, num_subcores=NS)
    fn = pl.kernel(
        _body,
        out_type=jax.ShapeDtypeStruct((N,), jnp.float32),
        mesh=mesh,
        scratch_types=[
            pltpu.VMEM((P + GB,), jnp.int32),
            pltpu.VMEM((FCAP + 16,), jnp.float32),
            pltpu.VMEM((GB,), jnp.int32),
            pltpu.VMEM((CHUNK,), jnp.float32),
            pltpu.VMEM((SB,), jnp.int32),
            pltpu.VMEM((SB,), jnp.int32),
            pltpu.VMEM((NCHT + 1 + 16,), jnp.int32),
            pltpu.SemaphoreType.DMA,
        ],
        compiler_params=pltpu.CompilerParams(needs_layout_passes=False),
        name="qtmask_sc",
    )
    return fn(x1d, r32, sk, vu, bounds)


def kernel(x, r_peaks, rand_vals):
    r32 = r_peaks.astype(jnp.int32)
    offsets = jnp.arange(-D1, D2, dtype=jnp.int32)
    pos = r32[:, None] + offsets[None, :]
    flat = (pos % N).reshape(-1)
    u = lax.iota(jnp.int32, UL)
    sk, su = lax.sort((flat, u), dimension=0, num_keys=1, is_stable=False)
    pk = su // L
    off = su % L
    posw = r32[pk] - D1 + off
    is_end = jnp.concatenate(
        [sk[1:] != sk[:-1], jnp.ones((1,), dtype=jnp.bool_)])
    keep = is_end & (posw < N) & (rand_vals[pk] > RATIO)
    vu = jnp.where(keep, pk, -1)
    grid = jnp.arange(0, N + 1, CHUNK, dtype=jnp.int32)
    bounds = jnp.searchsorted(sk, grid).astype(jnp.int32)
    out = _qtmask_sc(x.reshape(N), r32, sk, vu, bounds)
    return out.reshape(1, N)


# speedup vs baseline: 11.4989x; 11.4989x over previous
import jax
import jax.numpy as jnp
from jax import lax
from jax.experimental import pallas as pl
from jax.experimental.pallas import tpu as pltpu
from jax.experimental.pallas import tpu_sc as plsc

N = 16_000_000
P = 32_000
L = 191
D1 = 25
D2 = 166
UL = P * L
RATIO = 0.5
NC, NS = 2, 16
NW = NC * NS
RANGE = N // NW
CHUNK = 10_000
NCH = RANGE // CHUNK
NCHT = NW * NCH
GB = 128
SB = 2048
FCAP = P + GB


def _sload(ref, i):
    return ref[pl.ds(i, 16)][0]


def _body(x_hbm, r_hbm, rvals_hbm, sk_hbm, enc_hbm, bounds_hbm, out_hbm,
          r_v, rvals_v, fill_v, idx_v, buf_v, sk_v, enc_v, bounds_v, sem):
    wid = lax.axis_index("s") * NC + lax.axis_index("c")
    base = wid * RANGE
    is_last = wid == NW - 1

    pltpu.sync_copy(r_hbm, r_v.at[pl.ds(0, P)])
    pltpu.sync_copy(rvals_hbm, rvals_v.at[pl.ds(0, P)])
    pltpu.sync_copy(bounds_hbm, bounds_v.at[pl.ds(0, NCHT + 1)])

    def first_gt(val):
        def step(_, lh):
            lo, hi = lh
            mid = jnp.minimum((lo + hi) // 2, P - 1)
            go = lo < hi
            pred = _sload(r_v, mid) > val
            lo2 = jnp.where(go & ~pred, mid + 1, lo)
            hi2 = jnp.where(go & pred, mid, hi)
            return lo2, hi2
        lo, _ = lax.fori_loop(0, 15, step, (jnp.int32(0), jnp.int32(P)))
        return lo

    p_lo = jnp.where(is_last, 0, first_gt(base - D2))
    p_hi = jnp.where(is_last, P, first_gt(base + RANGE + D1 - 1))
    wbase = (p_lo // GB) * GB

    nbatch = (p_hi - wbase + GB - 1) // GB

    def gbatch(b, _):
        off = wbase + b * GB
        for g in range(GB // 16):
            rg = r_v[pl.ds(off + g * 16, 16)]
            fi = rg - 26 + jnp.where(rg < 26, N, 0)
            fi = jnp.where(rg == 25, 0, fi)
            idx_v[pl.ds(g * 16, 16)] = jnp.clip(fi, 0, N - 1)
        pltpu.async_copy(x_hbm.at[idx_v], fill_v.at[pl.ds(b * GB, GB)], sem).wait()
        return 0

    lax.fori_loop(0, nbatch, gbatch, 0)

    lane0 = lax.iota(jnp.int32, 16)

    def chunk_body(c, _):
        cb = base + c * CHUNK
        ce = cb + CHUNK
        pltpu.sync_copy(x_hbm.at[pl.ds(cb, CHUNK)], buf_v)

        gcid = wid * NCH + c
        us = _sload(bounds_v, gcid)
        ue = _sload(bounds_v, gcid + 1)
        us0 = us - us % 8
        nsb = (ue - us0 + SB - 1) // SB

        def sbatch(b, _):
            s_off = jnp.minimum(us0 + b * SB, UL - SB)
            s_off = pl.multiple_of(s_off, 8)
            cp1 = pltpu.async_copy(sk_hbm.at[pl.ds(s_off, SB + 16)], sk_v, sem)
            cp2 = pltpu.async_copy(enc_hbm.at[pl.ds(s_off, SB + 16)], enc_v, sem)
            cp1.wait()
            cp2.wait()
            napply = jnp.clip((ue - s_off + 15) // 16, 0, SB // 16)

            def apply16(t, _):
                k16 = sk_v[pl.ds(t * 16, 16)]
                kn16 = sk_v[pl.ds(t * 16 + 1, 16)]
                e16 = enc_v[pl.ds(t * 16, 16)]
                pk = lax.shift_right_logical(e16, 8)
                off = lax.bitwise_and(e16, 255)
                rpk = plsc.load_gather(r_v, [pk])
                rvpk = plsc.load_gather(rvals_v, [pk])
                m = ((k16 != kn16) & (rpk - D1 + off < N) & (rvpk > RATIO)
                     & (k16 >= cb) & (k16 < ce))
                fi = jnp.clip(pk - wbase, 0, FCAP - 1)
                vals = plsc.load_gather(fill_v, [fi])
                plsc.store_scatter(buf_v, [jnp.where(m, k16 - cb, lane0)], vals, mask=m)
                return 0

            lax.fori_loop(0, napply, apply16, 0)
            return 0

        lax.fori_loop(0, nsb, sbatch, 0)

        pltpu.sync_copy(buf_v, out_hbm.at[pl.ds(cb, CHUNK)])
        return 0

    lax.fori_loop(0, NCH, chunk_body, 0)


@jax.jit
def _qtmask_sc(x1d, r32, rand_vals, sk, enc, bounds):
    mesh = plsc.VectorSubcoreMesh(core_axis_name="c", subcore_axis_name="s",
                                  num_cores=NC, num_subcores=NS)
    fn = pl.kernel(
        _body,
        out_type=jax.ShapeDtypeStruct((N,), jnp.float32),
        mesh=mesh,
        scratch_types=[
            pltpu.VMEM((P + GB,), jnp.int32),
            pltpu.VMEM((P + 16,), jnp.float32),
            pltpu.VMEM((FCAP + 16,), jnp.float32),
            pltpu.VMEM((GB,), jnp.int32),
            pltpu.VMEM((CHUNK,), jnp.float32),
            pltpu.VMEM((SB + 16,), jnp.int32),
            pltpu.VMEM((SB + 16,), jnp.int32),
            pltpu.VMEM((NCHT + 1 + 16,), jnp.int32),
            pltpu.SemaphoreType.DMA,
        ],
        compiler_params=pltpu.CompilerParams(needs_layout_passes=False),
        name="qtmask_sc",
    )
    return fn(x1d, r32, rand_vals, sk, enc, bounds)


def kernel(x, r_peaks, rand_vals):
    r32 = r_peaks.astype(jnp.int32)
    offsets = jnp.arange(-D1, D2, dtype=jnp.int32)
    pos = r32[:, None] + offsets[None, :]
    flat = (pos % N).reshape(-1)
    pk2 = lax.broadcasted_iota(jnp.int32, (P, L), 0)
    off2 = lax.broadcasted_iota(jnp.int32, (P, L), 1)
    enc0 = (lax.shift_left(pk2, 8) | off2).reshape(-1)
    sk, enc = lax.sort((flat, enc0), dimension=0, num_keys=1, is_stable=False)
    grid = jnp.arange(0, N + 1, CHUNK, dtype=jnp.int32)
    bounds = jnp.searchsorted(sk, grid).astype(jnp.int32)
    skp = jnp.pad(sk, (0, 16), constant_values=N + 7)
    encp = jnp.pad(enc, (0, 16), constant_values=0)
    out = _qtmask_sc(x.reshape(N), r32, rand_vals, skp, encp, bounds)
    return out.reshape(1, N)


# speedup vs baseline: 11.5517x; 1.0046x over previous
import jax
import jax.numpy as jnp
from jax import lax
from jax.experimental import pallas as pl
from jax.experimental.pallas import tpu as pltpu
from jax.experimental.pallas import tpu_sc as plsc

N = 16_000_000
P = 32_000
L = 191
D1 = 25
D2 = 166
UL = P * L
RATIO = 0.5
NC, NS = 2, 16
NW = NC * NS
RANGE = N // NW
CHUNK = 10_000
NCH = RANGE // CHUNK
NCHT = NW * NCH
GB = 128
SB = 2048
FCAP = P + GB


def _sload(ref, i):
    return ref[pl.ds(i, 16)][0]


def _body(x_hbm, r_hbm, sk_hbm, enc_hbm, bounds_hbm, out_hbm,
          r_v, fill_v, idx_v, buf_v, sk_v, enc_v, bounds_v, sem):
    wid = lax.axis_index("s") * NC + lax.axis_index("c")
    base = wid * RANGE
    is_last = wid == NW - 1

    pltpu.sync_copy(r_hbm, r_v.at[pl.ds(0, P)])
    pltpu.sync_copy(bounds_hbm, bounds_v.at[pl.ds(0, NCHT + 1)])

    def first_gt(val):
        def step(_, lh):
            lo, hi = lh
            mid = jnp.minimum((lo + hi) // 2, P - 1)
            go = lo < hi
            pred = _sload(r_v, mid) > val
            lo2 = jnp.where(go & ~pred, mid + 1, lo)
            hi2 = jnp.where(go & pred, mid, hi)
            return lo2, hi2
        lo, _ = lax.fori_loop(0, 15, step, (jnp.int32(0), jnp.int32(P)))
        return lo

    p_lo = jnp.where(is_last, 0, first_gt(base - D2))
    p_hi = jnp.where(is_last, P, first_gt(base + RANGE + D1 - 1))
    wbase = (p_lo // GB) * GB

    nbatch = (p_hi - wbase + GB - 1) // GB

    def gbatch(b, _):
        off = wbase + b * GB
        for g in range(GB // 16):
            rg = r_v[pl.ds(off + g * 16, 16)]
            fi = rg - 26 + jnp.where(rg < 26, N, 0)
            fi = jnp.where(rg == 25, 0, fi)
            idx_v[pl.ds(g * 16, 16)] = jnp.clip(fi, 0, N - 1)
        pltpu.async_copy(x_hbm.at[idx_v], fill_v.at[pl.ds(b * GB, GB)], sem).wait()
        return 0

    lax.fori_loop(0, nbatch, gbatch, 0)

    lane0 = lax.iota(jnp.int32, 16)

    def chunk_body(c, _):
        cb = base + c * CHUNK
        ce = cb + CHUNK
        pltpu.sync_copy(x_hbm.at[pl.ds(cb, CHUNK)], buf_v)

        gcid = wid * NCH + c
        us = _sload(bounds_v, gcid)
        ue = _sload(bounds_v, gcid + 1)
        us0 = us - us % 8
        nsb = (ue - us0 + SB - 1) // SB

        def sbatch(b, _):
            s_off = jnp.minimum(us0 + b * SB, UL - SB)
            s_off = pl.multiple_of(s_off, 8)
            cp1 = pltpu.async_copy(sk_hbm.at[pl.ds(s_off, SB + 16)], sk_v, sem)
            cp2 = pltpu.async_copy(enc_hbm.at[pl.ds(s_off, SB + 16)], enc_v, sem)
            cp1.wait()
            cp2.wait()
            napply = jnp.clip((ue - s_off + 15) // 16, 0, SB // 16)

            def apply16(t, _):
                k16 = sk_v[pl.ds(t * 16, 16)]
                kn16 = sk_v[pl.ds(t * 16 + 1, 16)]
                e16 = enc_v[pl.ds(t * 16, 16)]
                pk = lax.shift_right_logical(lax.bitwise_and(e16, (1 << 23) - 1), 8)
                m = ((k16 != kn16) & (e16 < (1 << 30))
                     & (k16 >= cb) & (k16 < ce))
                fi = jnp.clip(pk - wbase, 0, FCAP - 1)
                vals = plsc.load_gather(fill_v, [fi])
                plsc.store_scatter(buf_v, [jnp.where(m, k16 - cb, lane0)], vals, mask=m)
                return 0

            lax.fori_loop(0, napply, apply16, 0)
            return 0

        lax.fori_loop(0, nsb, sbatch, 0)

        pltpu.sync_copy(buf_v, out_hbm.at[pl.ds(cb, CHUNK)])
        return 0

    lax.fori_loop(0, NCH, chunk_body, 0)


@jax.jit
def _qtmask_sc(x1d, r32, sk, enc, bounds):
    mesh = plsc.VectorSubcoreMesh(core_axis_name="c", subcore_axis_name="s",
                                  num_cores=NC, num_subcores=NS)
    fn = pl.kernel(
        _body,
        out_type=jax.ShapeDtypeStruct((N,), jnp.float32),
        mesh=mesh,
        scratch_types=[
            pltpu.VMEM((P + GB,), jnp.int32),
            pltpu.VMEM((FCAP + 16,), jnp.float32),
            pltpu.VMEM((GB,), jnp.int32),
            pltpu.VMEM((CHUNK,), jnp.float32),
            pltpu.VMEM((SB + 16,), jnp.int32),
            pltpu.VMEM((SB + 16,), jnp.int32),
            pltpu.VMEM((NCHT + 1 + 16,), jnp.int32),
            pltpu.SemaphoreType.DMA,
        ],
        compiler_params=pltpu.CompilerParams(needs_layout_passes=False),
        name="qtmask_sc",
    )
    return fn(x1d, r32, sk, enc, bounds)


def kernel(x, r_peaks, rand_vals):
    r32 = r_peaks.astype(jnp.int32)
    offsets = jnp.arange(-D1, D2, dtype=jnp.int32)
    pos = r32[:, None] + offsets[None, :]
    flat = (pos % N).reshape(-1)
    pk2 = lax.broadcasted_iota(jnp.int32, (P, L), 0)
    off2 = lax.broadcasted_iota(jnp.int32, (P, L), 1)
    lose = jnp.where((rand_vals[:, None] > RATIO) & (pos < N), 0, 1 << 30)
    enc0 = (lax.shift_left(pk2, 8) | off2 | lose).reshape(-1)
    sk, enc = lax.sort((flat, enc0), dimension=0, num_keys=1, is_stable=False)
    grid = jnp.arange(0, N + 1, CHUNK, dtype=jnp.int32)
    bounds = jnp.searchsorted(sk, grid).astype(jnp.int32)
    skp = jnp.pad(sk, (0, 16), constant_values=N + 7)
    encp = jnp.pad(enc, (0, 16), constant_values=0)
    out = _qtmask_sc(x.reshape(N), r32, skp, encp, bounds)
    return out.reshape(1, N)


# speedup vs baseline: 12.2461x; 1.0601x over previous
import jax
import jax.numpy as jnp
from jax import lax
from jax.experimental import pallas as pl
from jax.experimental.pallas import tpu as pltpu
from jax.experimental.pallas import tpu_sc as plsc

N = 16_000_000
P = 32_000
L = 191
D1 = 25
D2 = 166
UL = P * L
RATIO = 0.5
NC, NS = 2, 16
NW = NC * NS
RANGE = N // NW
CHUNK = 10_000
NCH = RANGE // CHUNK
NCHT = NW * NCH
GB = 128
SB = 2048
FCAP = P + GB


def _sload(ref, i):
    return ref[pl.ds(i, 16)][0]


def _body(x_hbm, r_hbm, sk_hbm, enc_hbm, bounds_hbm, out_hbm,
          r_v, fill_v, idx_v, buf_v, sk_v, enc_v, bounds_v, sem):
    wid = lax.axis_index("s") * NC + lax.axis_index("c")
    base = wid * RANGE
    is_last = wid == NW - 1

    pltpu.sync_copy(r_hbm, r_v.at[pl.ds(0, P)])
    pltpu.sync_copy(bounds_hbm, bounds_v.at[pl.ds(0, NCHT + 1)])

    def first_gt(val):
        def step(_, lh):
            lo, hi = lh
            mid = jnp.minimum((lo + hi) // 2, P - 1)
            go = lo < hi
            pred = _sload(r_v, mid) > val
            lo2 = jnp.where(go & ~pred, mid + 1, lo)
            hi2 = jnp.where(go & pred, mid, hi)
            return lo2, hi2
        lo, _ = lax.fori_loop(0, 15, step, (jnp.int32(0), jnp.int32(P)))
        return lo

    p_lo = jnp.where(is_last, 0, first_gt(base - D2))
    p_hi = jnp.where(is_last, P, first_gt(base + RANGE + D1 - 1))
    wbase = (p_lo // GB) * GB

    nbatch = (p_hi - wbase + GB - 1) // GB

    def gbatch(b, _):
        off = wbase + b * GB
        for g in range(GB // 16):
            rg = r_v[pl.ds(off + g * 16, 16)]
            fi = rg - 26 + jnp.where(rg < 26, N, 0)
            fi = jnp.where(rg == 25, 0, fi)
            idx_v[pl.ds(g * 16, 16)] = jnp.clip(fi, 0, N - 1)
        pltpu.async_copy(x_hbm.at[idx_v], fill_v.at[pl.ds(b * GB, GB)], sem).wait()
        return 0

    lax.fori_loop(0, nbatch, gbatch, 0)

    lane0 = lax.iota(jnp.int32, 16)

    def chunk_body(c, _):
        cb = base + c * CHUNK
        ce = cb + CHUNK
        pltpu.sync_copy(x_hbm.at[pl.ds(cb, CHUNK)], buf_v)

        gcid = wid * NCH + c
        us = jnp.maximum(_sload(bounds_v, gcid) - 64, 0)
        ue = jnp.minimum(_sload(bounds_v, gcid + 1) + 64, UL)
        us0 = us - us % 8
        nsb = (ue - us0 + SB - 1) // SB

        def sbatch(b, _):
            s_off = jnp.minimum(us0 + b * SB, UL - SB)
            s_off = pl.multiple_of(s_off, 8)
            cp1 = pltpu.async_copy(sk_hbm.at[pl.ds(s_off, SB + 16)], sk_v, sem)
            cp2 = pltpu.async_copy(enc_hbm.at[pl.ds(s_off, SB + 16)], enc_v, sem)
            cp1.wait()
            cp2.wait()
            napply = jnp.clip((ue - s_off + 15) // 16, 0, SB // 16)

            def apply16(t, _):
                k16 = sk_v[pl.ds(t * 16, 16)]
                kn16 = sk_v[pl.ds(t * 16 + 1, 16)]
                e16 = enc_v[pl.ds(t * 16, 16)]
                pk = lax.shift_right_logical(lax.bitwise_and(e16, (1 << 23) - 1), 8)
                m = ((k16 != kn16) & (e16 < (1 << 30))
                     & (k16 >= cb) & (k16 < ce))
                fi = jnp.clip(pk - wbase, 0, FCAP - 1)
                vals = plsc.load_gather(fill_v, [fi])
                plsc.store_scatter(buf_v, [jnp.where(m, k16 - cb, lane0)], vals, mask=m)
                return 0

            lax.fori_loop(0, napply, apply16, 0)
            return 0

        lax.fori_loop(0, nsb, sbatch, 0)

        pltpu.sync_copy(buf_v, out_hbm.at[pl.ds(cb, CHUNK)])
        return 0

    lax.fori_loop(0, NCH, chunk_body, 0)


@jax.jit
def _qtmask_sc(x1d, r32, sk, enc, bounds):
    mesh = plsc.VectorSubcoreMesh(core_axis_name="c", subcore_axis_name="s",
                                  num_cores=NC, num_subcores=NS)
    fn = pl.kernel(
        _body,
        out_type=jax.ShapeDtypeStruct((N,), jnp.float32),
        mesh=mesh,
        scratch_types=[
            pltpu.VMEM((P + GB,), jnp.int32),
            pltpu.VMEM((FCAP + 16,), jnp.float32),
            pltpu.VMEM((GB,), jnp.int32),
            pltpu.VMEM((CHUNK,), jnp.float32),
            pltpu.VMEM((SB + 16,), jnp.int32),
            pltpu.VMEM((SB + 16,), jnp.int32),
            pltpu.VMEM((NCHT + 1 + 16,), jnp.int32),
            pltpu.SemaphoreType.DMA,
        ],
        compiler_params=pltpu.CompilerParams(needs_layout_passes=False),
        name="qtmask_sc",
    )
    return fn(x1d, r32, sk, enc, bounds)


def kernel(x, r_peaks, rand_vals):
    r32 = r_peaks.astype(jnp.int32)
    offsets = jnp.arange(-D1, D2, dtype=jnp.int32)
    pos = r32[:, None] + offsets[None, :]
    flat = (pos % N).reshape(-1)
    pk2 = lax.broadcasted_iota(jnp.int32, (P, L), 0)
    off2 = lax.broadcasted_iota(jnp.int32, (P, L), 1)
    lose = jnp.where((rand_vals[:, None] > RATIO) & (pos < N), 0, 1 << 30)
    enc0 = (lax.shift_left(pk2, 8) | off2 | lose).reshape(-1)
    sk, enc = lax.sort((flat, enc0), dimension=0, num_keys=1, is_stable=False)
    a = jnp.maximum(r32 - D1, 0)
    b = jnp.minimum(r32 + D2, N)
    csum_len = jnp.concatenate([jnp.zeros((1,), jnp.int32),
                                jnp.cumsum(b - a, dtype=jnp.int32)])
    csum_a = jnp.concatenate([jnp.zeros((1,), jnp.int32),
                              jnp.cumsum(a, dtype=jnp.int32)])
    hi_wrap = jnp.sum(jnp.maximum(r32 + D2 - N, 0), dtype=jnp.int32)
    grid = jnp.arange(0, N + 1, CHUNK, dtype=jnp.int32)
    i1 = jnp.searchsorted(b, grid, side="right").astype(jnp.int32)
    i2 = jnp.searchsorted(a, grid, side="left").astype(jnp.int32)
    bounds = (csum_len[i1] + grid * (i2 - i1) - (csum_a[i2] - csum_a[i1])
              + hi_wrap)
    bounds = bounds.at[0].set(0).at[-1].set(UL)
    skp = jnp.pad(sk, (0, 16), constant_values=N + 7)
    encp = jnp.pad(enc, (0, 16), constant_values=0)
    out = _qtmask_sc(x.reshape(N), r32, skp, encp, bounds)
    return out.reshape(1, N)
